# Initial kernel scaffold; baseline (speedup 1.0000x reference)
import jax
import jax.numpy as jnp
from jax.experimental import pallas as pl


def kernel(x_cat, x_num, lin_table, v_table, W_num, b_num, bias):
    raise NotImplementedError("write your pallas kernel here")


# speedup vs baseline: 1.1644x; 1.1644x over previous
import functools

import jax
import jax.numpy as jnp
from jax import lax
from jax.experimental import pallas as pl
from jax.experimental.pallas import tpu as pltpu
from jax.experimental.pallas import tpu_sc as plsc

B = 16384
F = 26
K = 16
NN = 49

NC = 2
NS = 16
NW = NC * NS
ROWS_W = B // NW
CB = 128
NCHUNK = ROWS_W // CB
IDXR = CB * F // 128
NG = CB // 16


def _fm_body(xcat_hbm, xnum_hbm, lin_hbm, v_hbm, w_hbm, out_hbm,
             idx_v, vrows, linrows, xnum_v, w_v, out_v, sem_v, sem_l):
    wid = lax.axis_index("s") * NC + lax.axis_index("c")
    pltpu.sync_copy(w_hbm, w_v)
    iota = lax.iota(jnp.int32, 16)
    lane26 = iota * F
    zero16 = jnp.zeros((16,), jnp.int32)
    wchunks = [w_v[pl.ds(c * 16, 16)] for c in range(4)]
    ws = [wchunks[j // 16][j % 16] for j in range(NN)]
    bconst = wchunks[NN // 16][NN % 16]

    def chunk_body(c, carry):
        b0 = wid * ROWS_W + c * CB
        pltpu.sync_copy(xcat_hbm.at[pl.ds(b0 * F, CB * F)], idx_v)
        copies = []
        for i in range(IDXR):
            copies.append(pltpu.async_copy(
                v_hbm.at[idx_v.at[pl.ds(i * 128, 128)]],
                vrows.at[pl.ds(i * 128, 128)], sem_v))
            copies.append(pltpu.async_copy(
                lin_hbm.at[idx_v.at[pl.ds(i * 128, 128)]],
                linrows.at[pl.ds(i * 128, 128)], sem_l))
        pltpu.sync_copy(xnum_hbm.at[pl.ds(b0, CB)], xnum_v)
        for cp in copies:
            cp.wait()

        def group_body(g, gcarry):
            base = g * (16 * F)
            rvs = [lane26 + (base + f) for f in range(F)]
            lint = jnp.zeros((16,), jnp.float32)
            for f in range(F):
                lint = lint + plsc.load_gather(linrows, [rvs[f]])

            def kbody(k, kc):
                sos, acc2 = kc
                kk = jnp.full((16,), k, jnp.int32)
                acc = jnp.zeros((16,), jnp.float32)
                for f in range(F):
                    val = plsc.load_gather(vrows, [rvs[f], kk])
                    acc = acc + val
                    acc2 = acc2 + val * val
                return (sos + acc * acc, acc2)

            z = jnp.zeros((16,), jnp.float32)
            sos, acc2 = lax.fori_loop(0, K, kbody, (z, z))
            tot = lint + 0.5 * (sos - acc2)
            rowvec = iota + g * 16
            for j in range(NN):
                xv = plsc.load_gather(xnum_v, [rowvec, jnp.full((16,), j, jnp.int32)])
                tot = tot + ws[j] * xv
            tot = tot + bconst
            out_v[pl.ds(g * 16, 16)] = tot
            return gcarry

        lax.fori_loop(0, NG, group_body, 0)
        pltpu.sync_copy(out_v, out_hbm.at[pl.ds(b0, CB)])
        return carry

    lax.fori_loop(0, NCHUNK, chunk_body, 0)


@functools.partial(jax.jit, static_argnames=())
def kernel(x_cat, x_num, lin_table, v_table, W_num, b_num, bias):
    xcat_flat = x_cat.reshape(B * F)
    wvec = jnp.concatenate([
        W_num.reshape(-1),
        (b_num + bias).reshape(-1),
        jnp.zeros((14,), jnp.float32),
    ])
    mesh = plsc.VectorSubcoreMesh(core_axis_name="c", subcore_axis_name="s",
                                  num_cores=NC, num_subcores=NS)
    out = pl.kernel(
        _fm_body,
        out_type=jax.ShapeDtypeStruct((B,), jnp.float32),
        mesh=mesh,
        compiler_params=pltpu.CompilerParams(needs_layout_passes=False,
                                             use_tc_tiling_on_sc=False),
        scratch_types=[
            pltpu.VMEM((CB * F,), jnp.int32),
            pltpu.VMEM((CB * F, K), jnp.float32),
            pltpu.VMEM((CB * F,), jnp.float32),
            pltpu.VMEM((CB, NN), jnp.float32),
            pltpu.VMEM((64,), jnp.float32),
            pltpu.VMEM((CB,), jnp.float32),
            pltpu.SemaphoreType.DMA,
            pltpu.SemaphoreType.DMA,
        ],
    )(xcat_flat, x_num, lin_table.reshape(-1), v_table, wvec)
    return out.reshape(B, 1)
